# baseline (speedup 1.0000x reference)
import functools

import jax
import jax.numpy as jnp
from jax import lax
from jax.experimental import pallas as pl
from jax.experimental.pallas import tpu as pltpu
from jax.experimental.pallas import tpu_sc as plsc

EMBED = 64
NBUF = 4
NUM_WORKERS = 32
CL0, CL1 = 104, 96


@functools.lru_cache(maxsize=None)
def _make_gather(batch: int, seq: int):
    assert seq == CL0 + CL1
    rows_per_worker = batch // NUM_WORKERS
    n_steps = rows_per_worker * 2
    n_groups = n_steps // NBUF
    mesh = plsc.VectorSubcoreMesh(core_axis_name="c", subcore_axis_name="s")

    def step_params(s):
        return s // 2, (s % 2) * CL0, CL0 if s % 2 == 0 else CL1

    @functools.partial(
        pl.kernel,
        mesh=mesh,
        out_type=jax.ShapeDtypeStruct((batch, seq, EMBED), jnp.float32),
        scratch_types=[
            pltpu.VMEM((rows_per_worker, seq), jnp.int32),
            pltpu.VMEM((NBUF, CL0, EMBED), jnp.float32),
            pltpu.SemaphoreType.DMA((NBUF,)),
        ],
        compiler_params=pltpu.CompilerParams(use_tc_tiling_on_sc=False),
    )
    def gather_kernel(idx_hbm, table_hbm, out_hbm, idx_v, rows_v, gsem):
        wid = lax.axis_index("s") * 2 + lax.axis_index("c")
        row0 = wid * rows_per_worker
        pltpu.sync_copy(idx_hbm.at[pl.ds(row0, rows_per_worker)], idx_v)

        def start(s_dyn, b, s0, cl):
            r = s_dyn // 2
            pltpu.async_copy(
                table_hbm.at[idx_v.at[r, pl.ds(s0, cl)]],
                rows_v.at[b, pl.ds(0, cl)],
                gsem.at[b],
            )

        def finish(s_dyn, b, s0, cl):
            r = s_dyn // 2
            pltpu.make_async_copy(
                table_hbm.at[idx_v.at[r, pl.ds(s0, cl)]],
                rows_v.at[b, pl.ds(0, cl)],
                gsem.at[b],
            ).wait()
            pltpu.sync_copy(
                rows_v.at[b, pl.ds(0, cl)],
                out_hbm.at[row0 + r, pl.ds(s0, cl)],
            )

        for b in range(NBUF):
            r, s0, cl = step_params(b)
            start(b, b, s0, cl)

        def group(g, carry):
            j0 = g * NBUF
            for b in range(NBUF):
                _, s0, cl = step_params(b)
                finish(j0 + b, b, s0, cl)
                start(j0 + b + NBUF, b, s0, cl)
            return carry

        lax.fori_loop(0, n_groups - 1, group, 0)

        j0 = (n_groups - 1) * NBUF
        for b in range(NBUF):
            _, s0, cl = step_params(b)
            finish(j0 + b, b, s0, cl)

    return gather_kernel


def kernel(x, table):
    batch, seq = x.shape
    return _make_gather(batch, seq)(x, table)


# speedup vs baseline: 1.2259x; 1.2259x over previous
import functools

import jax
import jax.numpy as jnp
from jax import lax
from jax.experimental import pallas as pl
from jax.experimental.pallas import tpu as pltpu
from jax.experimental.pallas import tpu_sc as plsc

EMBED = 64
PADE = 128
NBUF = 4
NUM_WORKERS = 32
CL0, CL1 = 104, 96


@functools.lru_cache(maxsize=None)
def _make_gather(batch: int, seq: int):
    assert seq == CL0 + CL1
    rows_per_worker = batch // NUM_WORKERS
    n_steps = rows_per_worker * 2
    n_groups = n_steps // NBUF
    mesh = plsc.VectorSubcoreMesh(core_axis_name="c", subcore_axis_name="s")

    def step_params(s):
        return s // 2, (s % 2) * CL0, CL0 if s % 2 == 0 else CL1

    @functools.partial(
        pl.kernel,
        mesh=mesh,
        out_type=jax.ShapeDtypeStruct((batch, seq, PADE), jnp.float32),
        scratch_types=[
            pltpu.VMEM((rows_per_worker, seq), jnp.int32),
            pltpu.VMEM((NBUF, CL0, PADE), jnp.float32),
            pltpu.SemaphoreType.DMA((NBUF,)),
        ],
        compiler_params=pltpu.CompilerParams(use_tc_tiling_on_sc=False),
    )
    def gather_kernel(idx_hbm, table_hbm, out_hbm, idx_v, rows_v, gsem):
        wid = lax.axis_index("s") * 2 + lax.axis_index("c")
        row0 = wid * rows_per_worker
        pltpu.sync_copy(idx_hbm.at[pl.ds(row0, rows_per_worker)], idx_v)

        def start(s_dyn, b, s0, cl):
            r = s_dyn // 2
            pltpu.async_copy(
                table_hbm.at[idx_v.at[r, pl.ds(s0, cl)]],
                rows_v.at[b, pl.ds(0, cl)],
                gsem.at[b],
            )

        def finish(s_dyn, b, s0, cl):
            r = s_dyn // 2
            pltpu.make_async_copy(
                table_hbm.at[idx_v.at[r, pl.ds(s0, cl)]],
                rows_v.at[b, pl.ds(0, cl)],
                gsem.at[b],
            ).wait()
            pltpu.sync_copy(
                rows_v.at[b, pl.ds(0, cl)],
                out_hbm.at[row0 + r, pl.ds(s0, cl)],
            )

        for b in range(NBUF):
            _, s0, cl = step_params(b)
            start(b, b, s0, cl)

        def group(g, carry):
            j0 = g * NBUF
            for b in range(NBUF):
                _, s0, cl = step_params(b)
                finish(j0 + b, b, s0, cl)
                start(j0 + b + NBUF, b, s0, cl)
            return carry

        lax.fori_loop(0, n_groups - 1, group, 0)

        j0 = (n_groups - 1) * NBUF
        for b in range(NBUF):
            _, s0, cl = step_params(b)
            finish(j0 + b, b, s0, cl)

    return gather_kernel


def kernel(x, table):
    batch, seq = x.shape
    table_pad = jnp.pad(table, ((0, 0), (0, PADE - EMBED)))
    out_pad = _make_gather(batch, seq)(x, table_pad)
    return out_pad[:, :, :EMBED]


# speedup vs baseline: 1.3230x; 1.0793x over previous
import functools

import jax
import jax.numpy as jnp
from jax import lax
from jax.experimental import pallas as pl
from jax.experimental.pallas import tpu as pltpu
from jax.experimental.pallas import tpu_sc as plsc

EMBED = 64
PADE = 128
NBUF = 4
NUM_WORKERS = 32
CL0, CL1 = 104, 96


@functools.lru_cache(maxsize=None)
def _make_gather(batch: int, seq: int):
    assert seq == CL0 + CL1
    rows_per_worker = batch // NUM_WORKERS
    n_steps = rows_per_worker * 2
    n_groups = n_steps // NBUF
    mesh = plsc.VectorSubcoreMesh(core_axis_name="c", subcore_axis_name="s")

    def step_params(s):
        return s // 2, (s % 2) * CL0, CL0 if s % 2 == 0 else CL1

    @functools.partial(
        pl.kernel,
        mesh=mesh,
        out_type=jax.ShapeDtypeStruct((batch, seq, PADE), jnp.float32),
        scratch_types=[
            pltpu.VMEM((rows_per_worker, seq), jnp.int32),
            pltpu.VMEM((NBUF, CL0, PADE), jnp.float32),
            pltpu.SemaphoreType.DMA((NBUF,)),
        ],
        compiler_params=pltpu.CompilerParams(use_tc_tiling_on_sc=False),
    )
    def gather_kernel(idx_hbm, table_hbm, out_hbm, idx_v, rows_v, gsem):
        wid = lax.axis_index("s") * 2 + lax.axis_index("c")
        row0 = wid * rows_per_worker
        pltpu.sync_copy(idx_hbm.at[pl.ds(row0, rows_per_worker)], idx_v)

        def start(s_dyn, b, s0, cl):
            r = s_dyn // 2
            pltpu.async_copy(
                table_hbm.at[idx_v.at[r, pl.ds(s0, cl)]],
                rows_v.at[b, pl.ds(0, cl)],
                gsem.at[b],
            )

        def finish(s_dyn, b, s0, cl):
            r = s_dyn // 2
            pltpu.make_async_copy(
                table_hbm.at[idx_v.at[r, pl.ds(s0, cl)]],
                rows_v.at[b, pl.ds(0, cl)],
                gsem.at[b],
            ).wait()
            pltpu.sync_copy(
                rows_v.at[b, pl.ds(0, cl), pl.ds(0, EMBED)],
                out_hbm.at[row0 + r, pl.ds(s0, cl), pl.ds(0, EMBED)],
            )

        for b in range(NBUF):
            _, s0, cl = step_params(b)
            start(b, b, s0, cl)

        def group(g, carry):
            j0 = g * NBUF
            for b in range(NBUF):
                _, s0, cl = step_params(b)
                finish(j0 + b, b, s0, cl)
                start(j0 + b + NBUF, b, s0, cl)
            return carry

        lax.fori_loop(0, n_groups - 1, group, 0)

        j0 = (n_groups - 1) * NBUF
        for b in range(NBUF):
            _, s0, cl = step_params(b)
            finish(j0 + b, b, s0, cl)

    return gather_kernel


def kernel(x, table):
    batch, seq = x.shape
    table_pad = jnp.pad(table, ((0, 0), (0, PADE - EMBED)))
    out_pad = _make_gather(batch, seq)(x, table_pad)
    return out_pad[:, :, :EMBED]
